# baseline (device time: 9883 ns/iter reference)
import jax
import jax.numpy as jnp
from jax import lax
from jax.experimental import pallas as pl
from jax.experimental.pallas import tpu as pltpu

G = 4


def kernel(x):
    _, m, n = x.shape
    q_rows = m // G

    def body(x_ref, out_ref, acc_ref, p0, p1, p2, send_sems, recv_sems):
        i = lax.axis_index("i")
        p = i & 3
        plane_base = i - p
        prs = [p0, p1, p2]

        def plane_peer(d):
            return plane_base + ((p + d) & 3)

        acc_ref[...] = x_ref[0].astype(jnp.bfloat16)

        barrier_sem = pltpu.get_barrier_semaphore()
        for d in (1, 2, 3):
            pl.semaphore_signal(
                barrier_sem, inc=1,
                device_id=(plane_peer(d),), device_id_type=pl.DeviceIdType.MESH,
            )
        pl.semaphore_wait(barrier_sem, 3)

        sends = []
        for d in (1, 2, 3):
            pt = (p + d) & 3
            r = pltpu.make_async_remote_copy(
                src_ref=acc_ref.at[pl.ds(pt * q_rows, q_rows)],
                dst_ref=prs[3 - d],
                send_sem=send_sems.at[d - 1],
                recv_sem=recv_sems.at[3 - d],
                device_id=(plane_peer(d),),
                device_id_type=pl.DeviceIdType.MESH,
            )
            r.start()
            sends.append(r)
        my_q = p * q_rows
        for slot in (2, 0, 1):
            rr = pltpu.make_async_remote_copy(
                src_ref=prs[slot], dst_ref=prs[slot],
                send_sem=send_sems.at[slot], recv_sem=recv_sems.at[slot],
                device_id=(i,), device_id_type=pl.DeviceIdType.MESH,
            )
            rr.wait_recv()
            acc_ref[pl.ds(my_q, q_rows)] = (
                acc_ref[pl.ds(my_q, q_rows)] + prs[slot][...]
            )
        for r in sends:
            r.wait_send()

        out_ref[...] = acc_ref[...].astype(jnp.float32)

    return pl.pallas_call(
        body,
        out_shape=jax.ShapeDtypeStruct((m, n), jnp.float32),
        in_specs=[pl.BlockSpec(memory_space=pltpu.VMEM)],
        out_specs=pl.BlockSpec(memory_space=pltpu.VMEM),
        scratch_shapes=[
            pltpu.VMEM((m, n), jnp.bfloat16),
            pltpu.VMEM((q_rows, n), jnp.bfloat16),
            pltpu.VMEM((q_rows, n), jnp.bfloat16),
            pltpu.VMEM((q_rows, n), jnp.bfloat16),
            pltpu.SemaphoreType.DMA((3,)),
            pltpu.SemaphoreType.DMA((3,)),
        ],
        compiler_params=pltpu.CompilerParams(collective_id=0),
    )(x)
